# baseline (speedup 1.0000x reference)
import jax
import jax.numpy as jnp
from jax import lax
from jax.experimental import pallas as pl
from jax.experimental.pallas import tpu as pltpu
from jax.experimental.pallas import tpu_sc as plsc

N = 10000
E = 160000
F = 256
H = 512
C = 128

NB = 25
BN = N // NB
BE = E // NB

BLK = 80

_DN = lax.GatherDimensionNumbers(offset_dims=(), collapsed_slice_dims=(0,),
                                 start_index_map=(0,))


def _lane_splat(vec16, i):
    idx = jnp.full((16, 1), i, jnp.int32)
    return lax.gather(vec16, idx, _DN, slice_sizes=(1,),
                      mode=lax.GatherScatterMode.PROMISE_IN_BOUNDS)


ZROWS = 624
ZLAST = N - 15 * ZROWS


def _sc_segsum(tables, src, dst, em, zeros, *, split_edges_by_core):
    nt = len(tables)
    mesh = plsc.VectorSubcoreMesh(core_axis_name="c", subcore_axis_name="s")

    def body(*refs):
        tabs = refs[:nt]
        src_h, dst_h, em_h, z_h, o0, o1 = refs[nt:nt + 6]
        srcq, emq = refs[nt + 6:nt + 8]
        rows = refs[nt + 8:nt + 12]
        dstb = refs[nt + 12:nt + 16]
        gsem = refs[nt + 16:nt + 20]
        dsem = refs[nt + 20:nt + 24]
        ssem = refs[nt + 24:nt + 28]
        srcsem, emsem = refs[nt + 28:nt + 30]
        agg = refs[nt + 30]
        c = lax.axis_index("c")
        s = lax.axis_index("s")
        if split_edges_by_core:
            ebase = c * 80000 + jnp.where(s < 10, s * 5120,
                                          51200 + (s - 10) * 4800)
            nquad = jnp.where(s < 10, 16, 15)
        else:
            ebase = jnp.where(s < 12, s * 9920, 119040 + (s - 12) * 10240)
            nquad = jnp.where(s < 12, 31, 32)
        ebase = pl.multiple_of(ebase, 8)
        zoff = pl.multiple_of(s * ZROWS, 8)

        @pl.when(s < 15)
        def _():
            pltpu.sync_copy(z_h.at[pl.ds(0, ZROWS)],
                            agg.at[pl.ds(zoff, ZROWS)])

        @pl.when(s == 15)
        def _():
            pltpu.sync_copy(z_h, agg.at[pl.ds(15 * ZROWS, ZLAST)])

        plsc.subcore_barrier()

        def scale(rowbuf, boff):
            for g in range(BLK // 16):
                em16 = emq[pl.ds(boff + g * 16, 16)]
                ew16 = 1.0 / (1.0 + jnp.exp(-em16))
                for i in range(16):
                    w = _lane_splat(ew16, i)
                    e = g * 16 + i
                    for k in range(8):
                        sl = pl.ds(k * 16, 16)
                        rowbuf[e, sl] = rowbuf[e, sl] * w

        def make_quadfn(tab):
            def quadfn(q, carry):
                qoff = pl.multiple_of(q * (4 * BLK), 8)
                sv = pltpu.async_copy(src_h.at[pl.ds(ebase + qoff, 4 * BLK)],
                                      srcq, srcsem)
                ev = pltpu.async_copy(em_h.at[pl.ds(ebase + qoff, 4 * BLK)],
                                      emq, emsem)
                dd = [pltpu.async_copy(
                    dst_h.at[pl.ds(ebase + qoff + b * BLK, BLK)],
                    dstb[b], dsem[b]) for b in range(4)]
                sv.wait()
                gs = [pltpu.async_copy(tab.at[srcq.at[pl.ds(b * BLK, BLK)]],
                                       rows[b], gsem[b]) for b in range(4)]
                ev.wait()
                ss = []
                for b in range(4):
                    gs[b].wait()
                    scale(rows[b], b * BLK)
                    dd[b].wait()
                    ss.append(pltpu.async_copy(rows[b], agg.at[dstb[b]],
                                               ssem[b], add=True))
                for b in range(4):
                    ss[b].wait()
                return carry
            return quadfn

        if nt == 2:
            @pl.when(c == 0)
            def _():
                lax.fori_loop(0, nquad, make_quadfn(tabs[0]), 0)

            @pl.when(c == 1)
            def _():
                lax.fori_loop(0, nquad, make_quadfn(tabs[1]), 0)
        else:
            lax.fori_loop(0, nquad, make_quadfn(tabs[0]), 0)

        plsc.subcore_barrier()

        def writeback(dst_ref):
            @pl.when(s < 15)
            def _():
                pltpu.sync_copy(agg.at[pl.ds(zoff, ZROWS)],
                                dst_ref.at[pl.ds(zoff, ZROWS)])

            @pl.when(s == 15)
            def _():
                pltpu.sync_copy(agg.at[pl.ds(15 * ZROWS, ZLAST)],
                                dst_ref.at[pl.ds(15 * ZROWS, ZLAST)])

        @pl.when(c == 0)
        def _():
            writeback(o0)

        @pl.when(c == 1)
        def _():
            writeback(o1)

    kern = pl.kernel(
        body,
        out_type=(jax.ShapeDtypeStruct((N, 128), jnp.float32),
                  jax.ShapeDtypeStruct((N, 128), jnp.float32)),
        mesh=mesh,
        scratch_types=[
            pltpu.VMEM((4 * BLK,), jnp.int32),
            pltpu.VMEM((4 * BLK,), jnp.float32),
            *[pltpu.VMEM((BLK, 128), jnp.float32) for _ in range(4)],
            *[pltpu.VMEM((BLK,), jnp.int32) for _ in range(4)],
            *[pltpu.SemaphoreType.DMA for _ in range(14)],
            pltpu.VMEM_SHARED((N, 128), jnp.float32),
        ],
    )
    return kern(*tables, src, dst, em, zeros)


def _t2_body(a0_ref, a1_ref, nf_ref, w1_ref, w2_ref, y_ref):
    nfs = jax.nn.sigmoid(nf_ref[0])
    a0 = a0_ref[...] * nfs[None, :128]
    a1 = a1_ref[...] * nfs[None, 128:]
    z = (jnp.dot(a0, w1_ref[:128, :], preferred_element_type=jnp.float32)
         + jnp.dot(a1, w1_ref[128:, :], preferred_element_type=jnp.float32))
    h1 = jnp.maximum(z, 0.0)
    y_ref[...] = jnp.dot(h1, w2_ref[...], preferred_element_type=jnp.float32)


def _stage2(a0, a1, node_feat_mask, W1, W2):
    return pl.pallas_call(
        _t2_body,
        grid=(NB,),
        in_specs=[
            pl.BlockSpec((BN, 128), lambda i: (i, 0)),
            pl.BlockSpec((BN, 128), lambda i: (i, 0)),
            pl.BlockSpec((1, F), lambda i: (0, 0)),
            pl.BlockSpec((F, H), lambda i: (0, 0)),
            pl.BlockSpec((H, C), lambda i: (0, 0)),
        ],
        out_specs=pl.BlockSpec((BN, C), lambda i: (i, 0)),
        out_shape=jax.ShapeDtypeStruct((N, C), jnp.float32),
    )(a0, a1, node_feat_mask, W1, W2)


def _t3_body(p0_ref, p1_ref, pred_ref, em_ref, nf_ref, acc_ref):
    i = pl.program_id(0)

    @pl.when(i == 0)
    def _():
        nfm = jax.nn.sigmoid(nf_ref[...])
        ent2 = (-nfm * jnp.log(nfm + 1e-15)
                - (1.0 - nfm) * jnp.log(1.0 - nfm + 1e-15))
        acc_ref[0, 0] = 0.5 * jnp.mean(nfm) + 0.2 * jnp.mean(ent2)

    logits = p0_ref[...] + p1_ref[...]
    probs = jax.nn.sigmoid(logits)
    eps = 1e-12
    p = jnp.clip(probs, eps, 1.0 - eps)
    pred = pred_ref[...]
    bce = jnp.sum(pred * jnp.log(p) + (1.0 - pred) * jnp.log(1.0 - p))
    m = jax.nn.sigmoid(em_ref[...])
    s_ew = jnp.sum(m)
    ent = -m * jnp.log(m + 1e-15) - (1.0 - m) * jnp.log(1.0 - m + 1e-15)
    s_ent = jnp.sum(ent)
    part = (-bce / (N * C)) + (0.01 / E) * s_ew + (0.5 / E) * s_ent
    acc_ref[0, 0] = acc_ref[0, 0] + part


def _stage3(p0, p1, pred_label, em2d, node_feat_mask):
    return pl.pallas_call(
        _t3_body,
        grid=(NB,),
        in_specs=[
            pl.BlockSpec((BN, C), lambda i: (i, 0)),
            pl.BlockSpec((BN, C), lambda i: (i, 0)),
            pl.BlockSpec((BN, C), lambda i: (i, 0)),
            pl.BlockSpec((1, 1, BE), lambda i: (i, 0, 0)),
            pl.BlockSpec((1, F), lambda i: (0, 0)),
        ],
        out_specs=pl.BlockSpec((1, 1), lambda i: (0, 0),
                               memory_space=pltpu.MemorySpace.SMEM),
        out_shape=jax.ShapeDtypeStruct((1, 1), jnp.float32),
    )(p0, p1, pred_label, em2d, node_feat_mask)


def kernel(x, edge_index, node_feat_mask, edge_mask, W1, W2, pred_label):
    src = edge_index[0]
    dst = edge_index[1]
    em2d = edge_mask.reshape(NB, 1, BE)
    zeros = jnp.zeros((ZLAST, 128), jnp.float32)
    x0 = x[:, :128]
    x1 = x[:, 128:]

    a0, a1 = _sc_segsum((x0, x1), src, dst, edge_mask, zeros,
                        split_edges_by_core=False)

    y = _stage2(a0, a1, node_feat_mask, W1, W2)

    p0, p1 = _sc_segsum((y,), src, dst, edge_mask, zeros,
                        split_edges_by_core=True)

    acc = _stage3(p0, p1, pred_label, em2d, node_feat_mask)
    return acc[0, 0]


# speedup vs baseline: 1.0630x; 1.0630x over previous
import jax
import jax.numpy as jnp
from jax import lax
from jax.experimental import pallas as pl
from jax.experimental.pallas import tpu as pltpu
from jax.experimental.pallas import tpu_sc as plsc

N = 10000
E = 160000
F = 256
H = 512
C = 128

NB = 25
BN = N // NB
BE = E // NB

BLK = 80

_DN = lax.GatherDimensionNumbers(offset_dims=(), collapsed_slice_dims=(0,),
                                 start_index_map=(0,))


def _lane_splat(vec16, i):
    idx = jnp.full((16, 1), i, jnp.int32)
    return lax.gather(vec16, idx, _DN, slice_sizes=(1,),
                      mode=lax.GatherScatterMode.PROMISE_IN_BOUNDS)


ZROWS = 624
ZLAST = N - 15 * ZROWS


def _sc_segsum(tables, src, dst, em, zeros, *, split_edges_by_core, blk):
    nt = len(tables)
    mesh = plsc.VectorSubcoreMesh(core_axis_name="c", subcore_axis_name="s")
    quad = 4 * blk
    stage_len = 5120 if split_edges_by_core else 10240

    def body(*refs):
        tabs = refs[:nt]
        src_h, dst_h, em_h, z_h, o0, o1 = refs[nt:nt + 6]
        srcv, emq = refs[nt + 6:nt + 8]
        rows = refs[nt + 8:nt + 12]
        dstb = refs[nt + 12:nt + 16]
        gsem = refs[nt + 16:nt + 20]
        dsem = refs[nt + 20:nt + 24]
        ssem = refs[nt + 24:nt + 28]
        emsem = refs[nt + 28]
        agg = refs[nt + 29]
        c = lax.axis_index("c")
        s = lax.axis_index("s")
        if split_edges_by_core:
            ebase = c * 80000 + s * 5120
            nquad = jnp.where(s < 15, 16, 10)
            tail_len = 3200
        else:
            ebase = jnp.where(s < 15, s * 9984, 149760)
            nquad = jnp.where(s < 15, 39, 40)
            tail_len = stage_len
        ebase = pl.multiple_of(ebase, 8)

        @pl.when(s < 15)
        def _():
            pltpu.sync_copy(src_h.at[pl.ds(ebase, stage_len)], srcv)

        @pl.when(s == 15)
        def _():
            pltpu.sync_copy(src_h.at[pl.ds(ebase, tail_len)],
                            srcv.at[pl.ds(0, tail_len)])

        zoff = pl.multiple_of(s * ZROWS, 8)

        @pl.when(s < 15)
        def _():
            pltpu.sync_copy(z_h.at[pl.ds(0, ZROWS)],
                            agg.at[pl.ds(zoff, ZROWS)])

        @pl.when(s == 15)
        def _():
            pltpu.sync_copy(z_h, agg.at[pl.ds(15 * ZROWS, ZLAST)])

        plsc.subcore_barrier()

        def scale(rowbuf, boff):
            for g in range(blk // 16):
                em16 = emq[pl.ds(boff + g * 16, 16)]
                ew16 = 1.0 / (1.0 + jnp.exp(-em16))
                for i in range(16):
                    w = _lane_splat(ew16, i)
                    e = g * 16 + i
                    for k in range(8):
                        sl = pl.ds(k * 16, 16)
                        rowbuf[e, sl] = rowbuf[e, sl] * w

        def make_quadfn(tab):
            def quadfn(q, carry):
                qoff = pl.multiple_of(q * quad, 8)
                gs = [pltpu.async_copy(
                    tab.at[srcv.at[pl.ds(qoff + b * blk, blk)]],
                    rows[b], gsem[b]) for b in range(4)]
                ev = pltpu.async_copy(em_h.at[pl.ds(ebase + qoff, quad)],
                                      emq, emsem)
                dd = [pltpu.async_copy(
                    dst_h.at[pl.ds(ebase + qoff + b * blk, blk)],
                    dstb[b], dsem[b]) for b in range(4)]
                ev.wait()
                ss = []
                for b in range(4):
                    gs[b].wait()
                    scale(rows[b], b * blk)
                    dd[b].wait()
                    ss.append(pltpu.async_copy(rows[b], agg.at[dstb[b]],
                                               ssem[b], add=True))
                for b in range(4):
                    ss[b].wait()
                return carry
            return quadfn

        if nt == 2:
            @pl.when(c == 0)
            def _():
                lax.fori_loop(0, nquad, make_quadfn(tabs[0]), 0)

            @pl.when(c == 1)
            def _():
                lax.fori_loop(0, nquad, make_quadfn(tabs[1]), 0)
        else:
            lax.fori_loop(0, nquad, make_quadfn(tabs[0]), 0)

        plsc.subcore_barrier()

        def writeback(dst_ref):
            @pl.when(s < 15)
            def _():
                pltpu.sync_copy(agg.at[pl.ds(zoff, ZROWS)],
                                dst_ref.at[pl.ds(zoff, ZROWS)])

            @pl.when(s == 15)
            def _():
                pltpu.sync_copy(agg.at[pl.ds(15 * ZROWS, ZLAST)],
                                dst_ref.at[pl.ds(15 * ZROWS, ZLAST)])

        @pl.when(c == 0)
        def _():
            writeback(o0)

        @pl.when(c == 1)
        def _():
            writeback(o1)

    kern = pl.kernel(
        body,
        out_type=(jax.ShapeDtypeStruct((N, 128), jnp.float32),
                  jax.ShapeDtypeStruct((N, 128), jnp.float32)),
        mesh=mesh,
        scratch_types=[
            pltpu.VMEM((stage_len,), jnp.int32),
            pltpu.VMEM((quad,), jnp.float32),
            *[pltpu.VMEM((blk, 128), jnp.float32) for _ in range(4)],
            *[pltpu.VMEM((blk,), jnp.int32) for _ in range(4)],
            *[pltpu.SemaphoreType.DMA for _ in range(13)],
            pltpu.VMEM_SHARED((N, 128), jnp.float32),
        ],
    )
    return kern(*tables, src, dst, em, zeros)


def _t2_body(a0_ref, a1_ref, nf_ref, w1_ref, w2_ref, y_ref):
    nfs = jax.nn.sigmoid(nf_ref[0])
    a0 = a0_ref[...] * nfs[None, :128]
    a1 = a1_ref[...] * nfs[None, 128:]
    z = (jnp.dot(a0, w1_ref[:128, :], preferred_element_type=jnp.float32)
         + jnp.dot(a1, w1_ref[128:, :], preferred_element_type=jnp.float32))
    h1 = jnp.maximum(z, 0.0)
    y_ref[...] = jnp.dot(h1, w2_ref[...], preferred_element_type=jnp.float32)


def _stage2(a0, a1, node_feat_mask, W1, W2):
    return pl.pallas_call(
        _t2_body,
        grid=(NB,),
        in_specs=[
            pl.BlockSpec((BN, 128), lambda i: (i, 0)),
            pl.BlockSpec((BN, 128), lambda i: (i, 0)),
            pl.BlockSpec((1, F), lambda i: (0, 0)),
            pl.BlockSpec((F, H), lambda i: (0, 0)),
            pl.BlockSpec((H, C), lambda i: (0, 0)),
        ],
        out_specs=pl.BlockSpec((BN, C), lambda i: (i, 0)),
        out_shape=jax.ShapeDtypeStruct((N, C), jnp.float32),
    )(a0, a1, node_feat_mask, W1, W2)


def _t3_body(p0_ref, p1_ref, pred_ref, em_ref, nf_ref, acc_ref):
    i = pl.program_id(0)

    @pl.when(i == 0)
    def _():
        nfm = jax.nn.sigmoid(nf_ref[...])
        ent2 = (-nfm * jnp.log(nfm + 1e-15)
                - (1.0 - nfm) * jnp.log(1.0 - nfm + 1e-15))
        acc_ref[0, 0] = 0.5 * jnp.mean(nfm) + 0.2 * jnp.mean(ent2)

    logits = p0_ref[...] + p1_ref[...]
    probs = jax.nn.sigmoid(logits)
    eps = 1e-12
    p = jnp.clip(probs, eps, 1.0 - eps)
    pred = pred_ref[...]
    bce = jnp.sum(pred * jnp.log(p) + (1.0 - pred) * jnp.log(1.0 - p))
    m = jax.nn.sigmoid(em_ref[...])
    s_ew = jnp.sum(m)
    ent = -m * jnp.log(m + 1e-15) - (1.0 - m) * jnp.log(1.0 - m + 1e-15)
    s_ent = jnp.sum(ent)
    part = (-bce / (N * C)) + (0.01 / E) * s_ew + (0.5 / E) * s_ent
    acc_ref[0, 0] = acc_ref[0, 0] + part


def _stage3(p0, p1, pred_label, em2d, node_feat_mask):
    return pl.pallas_call(
        _t3_body,
        grid=(NB,),
        in_specs=[
            pl.BlockSpec((BN, C), lambda i: (i, 0)),
            pl.BlockSpec((BN, C), lambda i: (i, 0)),
            pl.BlockSpec((BN, C), lambda i: (i, 0)),
            pl.BlockSpec((1, 1, BE), lambda i: (i, 0, 0)),
            pl.BlockSpec((1, F), lambda i: (0, 0)),
        ],
        out_specs=pl.BlockSpec((1, 1), lambda i: (0, 0),
                               memory_space=pltpu.MemorySpace.SMEM),
        out_shape=jax.ShapeDtypeStruct((1, 1), jnp.float32),
    )(p0, p1, pred_label, em2d, node_feat_mask)


def kernel(x, edge_index, node_feat_mask, edge_mask, W1, W2, pred_label):
    src = edge_index[0]
    dst = edge_index[1]
    em2d = edge_mask.reshape(NB, 1, BE)
    zeros = jnp.zeros((ZLAST, 128), jnp.float32)
    x0 = x[:, :128]
    x1 = x[:, 128:]

    a0, a1 = _sc_segsum((x0, x1), src, dst, edge_mask, zeros,
                        split_edges_by_core=False, blk=64)

    y = _stage2(a0, a1, node_feat_mask, W1, W2)

    p0, p1 = _sc_segsum((y,), src, dst, edge_mask, zeros,
                        split_edges_by_core=True, blk=80)

    acc = _stage3(p0, p1, pred_label, em2d, node_feat_mask)
    return acc[0, 0]


# speedup vs baseline: 1.0969x; 1.0319x over previous
import jax
import jax.numpy as jnp
from jax import lax
from jax.experimental import pallas as pl
from jax.experimental.pallas import tpu as pltpu
from jax.experimental.pallas import tpu_sc as plsc

N = 10000
E = 160000
F = 256
H = 512
C = 128

NB = 25
BN = N // NB
BE = E // NB

BLK = 80

_DN = lax.GatherDimensionNumbers(offset_dims=(), collapsed_slice_dims=(0,),
                                 start_index_map=(0,))


def _lane_splat(vec16, i):
    idx = jnp.full((16, 1), i, jnp.int32)
    return lax.gather(vec16, idx, _DN, slice_sizes=(1,),
                      mode=lax.GatherScatterMode.PROMISE_IN_BOUNDS)


ZROWS = 624
ZLAST = N - 15 * ZROWS


def _sc_segsum(tables, src, dst, em, zeros, *, split_edges_by_core):
    nt = len(tables)
    mesh = plsc.VectorSubcoreMesh(core_axis_name="c", subcore_axis_name="s")
    if split_edges_by_core:
        stage_len = 5120
    else:
        stage_len = 10080

    def body(*refs):
        tabs = refs[:nt]
        src_h, dst_h, em_h, z_h, o0, o1 = refs[nt:nt + 6]
        (srcv, emv, rows_a, rows_b, dst_a, dst_b,
         gsem_a, gsem_b, dsem_a, dsem_b, ssem_a, ssem_b, agg) = refs[nt + 6:]
        c = lax.axis_index("c")
        s = lax.axis_index("s")
        if split_edges_by_core:
            ebase = c * 80000 + jnp.where(s < 12, s * 4960,
                                          59520 + (s - 12) * 5120)
            npair = jnp.where(s < 12, 31, 32)
        else:
            ebase = jnp.where(s < 8, s * 9920, 79360 + (s - 8) * 10080)
            npair = jnp.where(s < 8, 62, 63)
        ebase = pl.multiple_of(ebase, 8)
        pltpu.sync_copy(src_h.at[pl.ds(ebase, stage_len)], srcv)
        pltpu.sync_copy(em_h.at[pl.ds(ebase, stage_len)], emv)
        zoff = pl.multiple_of(s * ZROWS, 8)

        @pl.when(s < 15)
        def _():
            pltpu.sync_copy(z_h.at[pl.ds(0, ZROWS)],
                            agg.at[pl.ds(zoff, ZROWS)])

        @pl.when(s == 15)
        def _():
            pltpu.sync_copy(z_h, agg.at[pl.ds(15 * ZROWS, ZLAST)])

        plsc.subcore_barrier()

        def scale(rowbuf, boff):
            for g in range(BLK // 16):
                em16 = emv[pl.ds(pl.multiple_of(boff + g * 16, 8), 16)]
                ew16 = 1.0 / (1.0 + jnp.exp(-em16))
                for i in range(16):
                    w = _lane_splat(ew16, i)
                    e = g * 16 + i
                    for k in range(8):
                        sl = pl.ds(k * 16, 16)
                        rowbuf[e, sl] = rowbuf[e, sl] * w

        def make_pairfn(tab):
            def pairfn(jj, carry):
                off0 = pl.multiple_of(jj * (2 * BLK), 8)
                off1 = pl.multiple_of(off0 + BLK, 8)
                g_a = pltpu.async_copy(tab.at[srcv.at[pl.ds(off0, BLK)]],
                                       rows_a, gsem_a)
                g_b = pltpu.async_copy(tab.at[srcv.at[pl.ds(off1, BLK)]],
                                       rows_b, gsem_b)
                d_a = pltpu.async_copy(dst_h.at[pl.ds(ebase + off0, BLK)],
                                       dst_a, dsem_a)
                d_b = pltpu.async_copy(dst_h.at[pl.ds(ebase + off1, BLK)],
                                       dst_b, dsem_b)
                g_a.wait()
                scale(rows_a, off0)
                d_a.wait()
                s_a = pltpu.async_copy(rows_a, agg.at[dst_a], ssem_a,
                                       add=True)
                g_b.wait()
                scale(rows_b, off1)
                d_b.wait()
                s_b = pltpu.async_copy(rows_b, agg.at[dst_b], ssem_b,
                                       add=True)
                s_a.wait()
                s_b.wait()
                return carry
            return pairfn

        if nt == 2:
            @pl.when(c == 0)
            def _():
                lax.fori_loop(0, npair, make_pairfn(tabs[0]), 0)

            @pl.when(c == 1)
            def _():
                lax.fori_loop(0, npair, make_pairfn(tabs[1]), 0)
        else:
            lax.fori_loop(0, npair, make_pairfn(tabs[0]), 0)

        plsc.subcore_barrier()

        def writeback(dst_ref):
            @pl.when(s < 15)
            def _():
                pltpu.sync_copy(agg.at[pl.ds(zoff, ZROWS)],
                                dst_ref.at[pl.ds(zoff, ZROWS)])

            @pl.when(s == 15)
            def _():
                pltpu.sync_copy(agg.at[pl.ds(15 * ZROWS, ZLAST)],
                                dst_ref.at[pl.ds(15 * ZROWS, ZLAST)])

        @pl.when(c == 0)
        def _():
            writeback(o0)

        @pl.when(c == 1)
        def _():
            writeback(o1)

    kern = pl.kernel(
        body,
        out_type=(jax.ShapeDtypeStruct((N, 128), jnp.float32),
                  jax.ShapeDtypeStruct((N, 128), jnp.float32)),
        mesh=mesh,
        scratch_types=[
            pltpu.VMEM((stage_len,), jnp.int32),
            pltpu.VMEM((stage_len,), jnp.float32),
            pltpu.VMEM((BLK, 128), jnp.float32),
            pltpu.VMEM((BLK, 128), jnp.float32),
            pltpu.VMEM((BLK,), jnp.int32),
            pltpu.VMEM((BLK,), jnp.int32),
            pltpu.SemaphoreType.DMA,
            pltpu.SemaphoreType.DMA,
            pltpu.SemaphoreType.DMA,
            pltpu.SemaphoreType.DMA,
            pltpu.SemaphoreType.DMA,
            pltpu.SemaphoreType.DMA,
            pltpu.VMEM_SHARED((N, 128), jnp.float32),
        ],
    )
    return kern(*tables, src, dst, em, zeros)


def _t2_body(a0_ref, a1_ref, nf_ref, w1_ref, w2_ref, y_ref):
    nfs = jax.nn.sigmoid(nf_ref[0])
    a0 = a0_ref[...] * nfs[None, :128]
    a1 = a1_ref[...] * nfs[None, 128:]
    z = (jnp.dot(a0, w1_ref[:128, :], preferred_element_type=jnp.float32)
         + jnp.dot(a1, w1_ref[128:, :], preferred_element_type=jnp.float32))
    h1 = jnp.maximum(z, 0.0)
    y_ref[...] = jnp.dot(h1, w2_ref[...], preferred_element_type=jnp.float32)


def _stage2(a0, a1, node_feat_mask, W1, W2):
    return pl.pallas_call(
        _t2_body,
        grid=(NB,),
        in_specs=[
            pl.BlockSpec((BN, 128), lambda i: (i, 0)),
            pl.BlockSpec((BN, 128), lambda i: (i, 0)),
            pl.BlockSpec((1, F), lambda i: (0, 0)),
            pl.BlockSpec((F, H), lambda i: (0, 0)),
            pl.BlockSpec((H, C), lambda i: (0, 0)),
        ],
        out_specs=pl.BlockSpec((BN, C), lambda i: (i, 0)),
        out_shape=jax.ShapeDtypeStruct((N, C), jnp.float32),
    )(a0, a1, node_feat_mask, W1, W2)


def _t3_body(p0_ref, p1_ref, pred_ref, em_ref, nf_ref, acc_ref):
    i = pl.program_id(0)

    @pl.when(i == 0)
    def _():
        nfm = jax.nn.sigmoid(nf_ref[...])
        ent2 = (-nfm * jnp.log(nfm + 1e-15)
                - (1.0 - nfm) * jnp.log(1.0 - nfm + 1e-15))
        acc_ref[0, 0] = 0.5 * jnp.mean(nfm) + 0.2 * jnp.mean(ent2)

    logits = p0_ref[...] + p1_ref[...]
    probs = jax.nn.sigmoid(logits)
    eps = 1e-12
    p = jnp.clip(probs, eps, 1.0 - eps)
    pred = pred_ref[...]
    bce = jnp.sum(pred * jnp.log(p) + (1.0 - pred) * jnp.log(1.0 - p))
    m = jax.nn.sigmoid(em_ref[...])
    s_ew = jnp.sum(m)
    ent = -m * jnp.log(m + 1e-15) - (1.0 - m) * jnp.log(1.0 - m + 1e-15)
    s_ent = jnp.sum(ent)
    part = (-bce / (N * C)) + (0.01 / E) * s_ew + (0.5 / E) * s_ent
    acc_ref[0, 0] = acc_ref[0, 0] + part


def _stage3(p0, p1, pred_label, em2d, node_feat_mask):
    return pl.pallas_call(
        _t3_body,
        grid=(NB,),
        in_specs=[
            pl.BlockSpec((BN, C), lambda i: (i, 0)),
            pl.BlockSpec((BN, C), lambda i: (i, 0)),
            pl.BlockSpec((BN, C), lambda i: (i, 0)),
            pl.BlockSpec((1, 1, BE), lambda i: (i, 0, 0)),
            pl.BlockSpec((1, F), lambda i: (0, 0)),
        ],
        out_specs=pl.BlockSpec((1, 1), lambda i: (0, 0),
                               memory_space=pltpu.MemorySpace.SMEM),
        out_shape=jax.ShapeDtypeStruct((1, 1), jnp.float32),
    )(p0, p1, pred_label, em2d, node_feat_mask)


def kernel(x, edge_index, node_feat_mask, edge_mask, W1, W2, pred_label):
    src = edge_index[0]
    dst = edge_index[1]
    em2d = edge_mask.reshape(NB, 1, BE)
    zeros = jnp.zeros((ZLAST, 128), jnp.float32)
    x0 = x[:, :128]
    x1 = x[:, 128:]

    a0, a1 = _sc_segsum((x0, x1), src, dst, edge_mask, zeros,
                        split_edges_by_core=False)

    y = _stage2(a0, a1, node_feat_mask, W1, W2)

    p0, p1 = _sc_segsum((y,), src, dst, edge_mask, zeros,
                        split_edges_by_core=True)

    acc = _stage3(p0, p1, pred_label, em2d, node_feat_mask)
    return acc[0, 0]


# speedup vs baseline: 1.0984x; 1.0014x over previous
import jax
import jax.numpy as jnp
from jax import lax
from jax.experimental import pallas as pl
from jax.experimental.pallas import tpu as pltpu
from jax.experimental.pallas import tpu_sc as plsc

N = 10000
E = 160000
F = 256
H = 512
C = 128

NB = 25
BN = N // NB
BE = E // NB

BLK = 80

_DN = lax.GatherDimensionNumbers(offset_dims=(), collapsed_slice_dims=(0,),
                                 start_index_map=(0,))


def _lane_splat(vec16, i):
    idx = jnp.full((16, 1), i, jnp.int32)
    return lax.gather(vec16, idx, _DN, slice_sizes=(1,),
                      mode=lax.GatherScatterMode.PROMISE_IN_BOUNDS)


ZROWS = 624
ZLAST = N - 15 * ZROWS


def _sc_segsum(tables, src, dst, em, zeros, *, split_edges_by_core,
               interleaved_table=False):
    nt = len(tables)
    mesh = plsc.VectorSubcoreMesh(core_axis_name="c", subcore_axis_name="s")
    if split_edges_by_core:
        stage_len = 5120
    else:
        stage_len = 10080

    def body(*refs):
        tabs = refs[:nt]
        src_h, dst_h, em_h, z_h, o0, o1 = refs[nt:nt + 6]
        (srcv, emv, rows_a, rows_b, dst_a, dst_b, adj_a, adj_b,
         gsem_a, gsem_b, dsem_a, dsem_b, ssem_a, ssem_b, agg) = refs[nt + 6:]
        c = lax.axis_index("c")
        s = lax.axis_index("s")
        if split_edges_by_core:
            ebase = c * 80000 + jnp.where(s < 12, s * 4960,
                                          59520 + (s - 12) * 5120)
            npair = jnp.where(s < 12, 31, 32)
        else:
            ebase = jnp.where(s < 8, s * 9920, 79360 + (s - 8) * 10080)
            npair = jnp.where(s < 8, 62, 63)
        ebase = pl.multiple_of(ebase, 8)
        pltpu.sync_copy(src_h.at[pl.ds(ebase, stage_len)], srcv)
        pltpu.sync_copy(em_h.at[pl.ds(ebase, stage_len)], emv)
        zoff = pl.multiple_of(s * ZROWS, 8)

        @pl.when(s < 15)
        def _():
            pltpu.sync_copy(z_h.at[pl.ds(0, ZROWS)],
                            agg.at[pl.ds(zoff, ZROWS)])

        @pl.when(s == 15)
        def _():
            pltpu.sync_copy(z_h, agg.at[pl.ds(15 * ZROWS, ZLAST)])

        plsc.subcore_barrier()

        def scale(rowbuf, boff):
            for g in range(BLK // 16):
                em16 = emv[pl.ds(pl.multiple_of(boff + g * 16, 8), 16)]
                ew16 = 1.0 / (1.0 + jnp.exp(-em16))
                for i in range(16):
                    w = _lane_splat(ew16, i)
                    e = g * 16 + i
                    for k in range(8):
                        sl = pl.ds(k * 16, 16)
                        rowbuf[e, sl] = rowbuf[e, sl] * w

        def make_pairfn(tab):
            def pairfn(jj, carry):
                off0 = pl.multiple_of(jj * (2 * BLK), 8)
                off1 = pl.multiple_of(off0 + BLK, 8)
                if interleaved_table:
                    for g in range(BLK // 16):
                        sl = pl.ds(g * 16, 16)
                        v = srcv[pl.ds(pl.multiple_of(off0 + g * 16, 8), 16)]
                        adj_a[sl] = v * 2 + c
                        v = srcv[pl.ds(pl.multiple_of(off1 + g * 16, 8), 16)]
                        adj_b[sl] = v * 2 + c
                    idx_a, idx_b = adj_a, adj_b
                else:
                    idx_a = srcv.at[pl.ds(off0, BLK)]
                    idx_b = srcv.at[pl.ds(off1, BLK)]
                g_a = pltpu.async_copy(tab.at[idx_a], rows_a, gsem_a)
                g_b = pltpu.async_copy(tab.at[idx_b], rows_b, gsem_b)
                d_a = pltpu.async_copy(dst_h.at[pl.ds(ebase + off0, BLK)],
                                       dst_a, dsem_a)
                d_b = pltpu.async_copy(dst_h.at[pl.ds(ebase + off1, BLK)],
                                       dst_b, dsem_b)
                g_a.wait()
                scale(rows_a, off0)
                d_a.wait()
                s_a = pltpu.async_copy(rows_a, agg.at[dst_a], ssem_a,
                                       add=True)
                g_b.wait()
                scale(rows_b, off1)
                d_b.wait()
                s_b = pltpu.async_copy(rows_b, agg.at[dst_b], ssem_b,
                                       add=True)
                s_a.wait()
                s_b.wait()
                return carry
            return pairfn

        lax.fori_loop(0, npair, make_pairfn(tabs[0]), 0)

        plsc.subcore_barrier()

        def writeback(dst_ref):
            @pl.when(s < 15)
            def _():
                pltpu.sync_copy(agg.at[pl.ds(zoff, ZROWS)],
                                dst_ref.at[pl.ds(zoff, ZROWS)])

            @pl.when(s == 15)
            def _():
                pltpu.sync_copy(agg.at[pl.ds(15 * ZROWS, ZLAST)],
                                dst_ref.at[pl.ds(15 * ZROWS, ZLAST)])

        @pl.when(c == 0)
        def _():
            writeback(o0)

        @pl.when(c == 1)
        def _():
            writeback(o1)

    kern = pl.kernel(
        body,
        out_type=(jax.ShapeDtypeStruct((N, 128), jnp.float32),
                  jax.ShapeDtypeStruct((N, 128), jnp.float32)),
        mesh=mesh,
        scratch_types=[
            pltpu.VMEM((stage_len,), jnp.int32),
            pltpu.VMEM((stage_len,), jnp.float32),
            pltpu.VMEM((BLK, 128), jnp.float32),
            pltpu.VMEM((BLK, 128), jnp.float32),
            pltpu.VMEM((BLK,), jnp.int32),
            pltpu.VMEM((BLK,), jnp.int32),
            pltpu.VMEM((BLK,), jnp.int32),
            pltpu.VMEM((BLK,), jnp.int32),
            pltpu.SemaphoreType.DMA,
            pltpu.SemaphoreType.DMA,
            pltpu.SemaphoreType.DMA,
            pltpu.SemaphoreType.DMA,
            pltpu.SemaphoreType.DMA,
            pltpu.SemaphoreType.DMA,
            pltpu.VMEM_SHARED((N, 128), jnp.float32),
        ],
    )
    return kern(*tables, src, dst, em, zeros)


def _t2_body(a0_ref, a1_ref, nf_ref, w1_ref, w2_ref, y_ref):
    nfs = jax.nn.sigmoid(nf_ref[0])
    a0 = a0_ref[...] * nfs[None, :128]
    a1 = a1_ref[...] * nfs[None, 128:]
    z = (jnp.dot(a0, w1_ref[:128, :], preferred_element_type=jnp.float32)
         + jnp.dot(a1, w1_ref[128:, :], preferred_element_type=jnp.float32))
    h1 = jnp.maximum(z, 0.0)
    y_ref[...] = jnp.dot(h1, w2_ref[...], preferred_element_type=jnp.float32)


def _stage2(a0, a1, node_feat_mask, W1, W2):
    return pl.pallas_call(
        _t2_body,
        grid=(NB,),
        in_specs=[
            pl.BlockSpec((BN, 128), lambda i: (i, 0)),
            pl.BlockSpec((BN, 128), lambda i: (i, 0)),
            pl.BlockSpec((1, F), lambda i: (0, 0)),
            pl.BlockSpec((F, H), lambda i: (0, 0)),
            pl.BlockSpec((H, C), lambda i: (0, 0)),
        ],
        out_specs=pl.BlockSpec((BN, C), lambda i: (i, 0)),
        out_shape=jax.ShapeDtypeStruct((N, C), jnp.float32),
    )(a0, a1, node_feat_mask, W1, W2)


def _t3_body(p0_ref, p1_ref, pred_ref, em_ref, nf_ref, acc_ref):
    i = pl.program_id(0)

    @pl.when(i == 0)
    def _():
        nfm = jax.nn.sigmoid(nf_ref[...])
        ent2 = (-nfm * jnp.log(nfm + 1e-15)
                - (1.0 - nfm) * jnp.log(1.0 - nfm + 1e-15))
        acc_ref[0, 0] = 0.5 * jnp.mean(nfm) + 0.2 * jnp.mean(ent2)

    logits = p0_ref[...] + p1_ref[...]
    probs = jax.nn.sigmoid(logits)
    eps = 1e-12
    p = jnp.clip(probs, eps, 1.0 - eps)
    pred = pred_ref[...]
    bce = jnp.sum(pred * jnp.log(p) + (1.0 - pred) * jnp.log(1.0 - p))
    m = jax.nn.sigmoid(em_ref[...])
    s_ew = jnp.sum(m)
    ent = -m * jnp.log(m + 1e-15) - (1.0 - m) * jnp.log(1.0 - m + 1e-15)
    s_ent = jnp.sum(ent)
    part = (-bce / (N * C)) + (0.01 / E) * s_ew + (0.5 / E) * s_ent
    acc_ref[0, 0] = acc_ref[0, 0] + part


def _stage3(p0, p1, pred_label, em2d, node_feat_mask):
    return pl.pallas_call(
        _t3_body,
        grid=(NB,),
        in_specs=[
            pl.BlockSpec((BN, C), lambda i: (i, 0)),
            pl.BlockSpec((BN, C), lambda i: (i, 0)),
            pl.BlockSpec((BN, C), lambda i: (i, 0)),
            pl.BlockSpec((1, 1, BE), lambda i: (i, 0, 0)),
            pl.BlockSpec((1, F), lambda i: (0, 0)),
        ],
        out_specs=pl.BlockSpec((1, 1), lambda i: (0, 0),
                               memory_space=pltpu.MemorySpace.SMEM),
        out_shape=jax.ShapeDtypeStruct((1, 1), jnp.float32),
    )(p0, p1, pred_label, em2d, node_feat_mask)


def kernel(x, edge_index, node_feat_mask, edge_mask, W1, W2, pred_label):
    src = edge_index[0]
    dst = edge_index[1]
    em2d = edge_mask.reshape(NB, 1, BE)
    zeros = jnp.zeros((ZLAST, 128), jnp.float32)
    x2 = x.reshape(2 * N, 128)

    a0, a1 = _sc_segsum((x2,), src, dst, edge_mask, zeros,
                        split_edges_by_core=False, interleaved_table=True)

    y = _stage2(a0, a1, node_feat_mask, W1, W2)

    p0, p1 = _sc_segsum((y,), src, dst, edge_mask, zeros,
                        split_edges_by_core=True)

    acc = _stage3(p0, p1, pred_label, em2d, node_feat_mask)
    return acc[0, 0]
